# Initial kernel scaffold; baseline (speedup 1.0000x reference)
import jax
import jax.numpy as jnp
from jax.experimental import pallas as pl


def kernel(x, edge_index, params, eps):
    raise NotImplementedError("write your pallas kernel here")


# speedup vs baseline: 42.8456x; 42.8456x over previous
import functools

import jax
import jax.numpy as jnp
from jax import lax
from jax.experimental import pallas as pl
from jax.experimental.pallas import tpu as pltpu
from jax.experimental.pallas import tpu_sc as plsc

N = 10000
D_IN = 128
HID = 16
LAT = 32
HEADS = 8
E = 320000

NP = 10240
SPT = NP // 16
ET = E + N
NW = 32
BW = 10496
ET_PAD = BW * NW
BR = 512
GRID = NP // BR


def _make_agg(F, C):
    nh = F // 16
    head_of = [(j * 16) // C for j in range(nh)]
    CH = 128 if F == 128 else 256
    NCH = BW // CH
    mesh = plsc.VectorSubcoreMesh(core_axis_name="c", subcore_axis_name="s")

    @functools.partial(
        pl.kernel,
        mesh=mesh,
        compiler_params=pltpu.CompilerParams(use_tc_tiling_on_sc=False),
        out_type=[
            jax.ShapeDtypeStruct((2, NP, F), jnp.float32),
            jax.ShapeDtypeStruct((2, NP, 16), jnp.float32),
        ],
        scratch_types=[
            pltpu.VMEM_SHARED((NP, F), jnp.float32),
            pltpu.VMEM_SHARED((NP, 16), jnp.float32),
            pltpu.VMEM((CH,), jnp.int32),
            pltpu.VMEM((CH,), jnp.int32),
            pltpu.VMEM((CH, 16), jnp.float32),
            pltpu.VMEM((CH, 16), jnp.float32),
            pltpu.VMEM((CH, 16), jnp.float32),
            pltpu.VMEM((CH, F), jnp.float32),
            pltpu.SemaphoreType.DMA,
            pltpu.SemaphoreType.DMA,
        ],
    )
    def agg(h_hbm, asrc_hbm, adst_hbm, src_hbm, dst_hbm, zf_hbm, zh_hbm,
            acc_hbm, den_hbm,
            acc_sh, den_sh, srcv, dstv, av, bv, aev, hv, sem_ab, sem_h):
        cid = lax.axis_index("c")
        sid = lax.axis_index("s")
        wid = sid * 2 + cid
        row0 = sid * SPT

        pltpu.sync_copy(zf_hbm, acc_sh.at[pl.ds(row0, SPT)])
        pltpu.sync_copy(zh_hbm, den_sh.at[pl.ds(row0, SPT)])
        plsc.subcore_barrier()

        wbase = wid * BW

        def alpha_body(e, _):
            al = av[e, :] + bv[e, :]
            al = jnp.where(al > 0.0, al, al * 0.2)
            aev[e, :] = jnp.exp(al)
            return 0

        def weight_body(e, _):
            ae = aev[e, :]
            for j in range(nh):
                s = ae[head_of[j]]
                hv[e, pl.ds(j * 16, 16)] = hv[e, pl.ds(j * 16, 16)] * s
            return 0

        def chunk_body(t, _):
            base = wbase + t * CH
            pltpu.sync_copy(src_hbm.at[pl.ds(base, CH)], srcv)
            pltpu.sync_copy(dst_hbm.at[pl.ds(base, CH)], dstv)
            cp_h = pltpu.async_copy(h_hbm.at[srcv], hv, sem_h)
            cp_a = pltpu.async_copy(asrc_hbm.at[srcv], av, sem_ab)
            cp_b = pltpu.async_copy(adst_hbm.at[dstv], bv, sem_ab)
            cp_a.wait()
            cp_b.wait()
            lax.fori_loop(0, CH, alpha_body, 0)
            cp_h.wait()
            lax.fori_loop(0, CH, weight_body, 0)
            pltpu.sync_copy(aev, den_sh.at[dstv], add=True)
            pltpu.sync_copy(hv, acc_sh.at[dstv], add=True)
            return 0

        lax.fori_loop(0, NCH, chunk_body, 0)

        plsc.subcore_barrier()
        pltpu.sync_copy(acc_sh.at[pl.ds(row0, SPT)],
                        acc_hbm.at[cid, pl.ds(row0, SPT)])
        pltpu.sync_copy(den_sh.at[pl.ds(row0, SPT)],
                        den_hbm.at[cid, pl.ds(row0, SPT)])

    return agg


_AGG128 = _make_agg(128, 16)
_AGG32 = _make_agg(32, 32)
_AGG128C = _make_agg(128, 128)


def _dot(a, b):
    return jax.lax.dot_general(a, b, (((1,), (0,)), ((), ())),
                               preferred_element_type=jnp.float32)


def _k1_body(x_ref, w_ref, s_ref, d_ref, h_ref, as_ref, ad_ref):
    h = _dot(x_ref[...], w_ref[...])
    h_ref[...] = h
    as_ref[...] = _dot(h, s_ref[...])
    ad_ref[...] = _dot(h, d_ref[...])


def _dense1(xp, w, s, d, f_out):
    return pl.pallas_call(
        _k1_body,
        grid=(GRID,),
        in_specs=[
            pl.BlockSpec((BR, xp.shape[1]), lambda i: (i, 0)),
            pl.BlockSpec(w.shape, lambda i: (0, 0)),
            pl.BlockSpec(s.shape, lambda i: (0, 0)),
            pl.BlockSpec(d.shape, lambda i: (0, 0)),
        ],
        out_specs=[
            pl.BlockSpec((BR, f_out), lambda i: (i, 0)),
            pl.BlockSpec((BR, 16), lambda i: (i, 0)),
            pl.BlockSpec((BR, 16), lambda i: (i, 0)),
        ],
        out_shape=[
            jax.ShapeDtypeStruct((NP, f_out), jnp.float32),
            jax.ShapeDtypeStruct((NP, 16), jnp.float32),
            jax.ShapeDtypeStruct((NP, 16), jnp.float32),
        ],
    )(xp, w, s, d)


def _finalize_block(acc_ref, den_ref, bias_ref, emat_ref, relu):
    acc = acc_ref[0] + acc_ref[1]
    den = den_ref[0] + den_ref[1]
    rcp = 1.0 / (den + 1e-16)
    out = acc * _dot(rcp, emat_ref[...]) + bias_ref[...]
    if relu:
        out = jnp.maximum(out, 0.0)
    return out


def _k2_body(acc_ref, den_ref, b_ref, e_ref,
             wm_ref, sm_ref, dm_ref, wl_ref, sl_ref, dl_ref,
             hm_ref, ams_ref, amd_ref, hl_ref, als_ref, ald_ref):
    hr = _finalize_block(acc_ref, den_ref, b_ref, e_ref, relu=True)
    hm = _dot(hr, wm_ref[...])
    hm_ref[...] = hm
    ams_ref[...] = _dot(hm, sm_ref[...])
    amd_ref[...] = _dot(hm, dm_ref[...])
    hl = _dot(hr, wl_ref[...])
    hl_ref[...] = hl
    als_ref[...] = _dot(hl, sl_ref[...])
    ald_ref[...] = _dot(hl, dl_ref[...])


def _k3_body(am_ref, dm_ref, al_ref, dl_ref, bm_ref, bl_ref, eps_ref,
             w3_ref, s3_ref, d3_ref,
             mu_ref, lv_ref, h3_ref, a3s_ref, a3d_ref):
    rm = 1.0 / (dm_ref[0, :, :1] + dm_ref[1, :, :1] + 1e-16)
    mu = (am_ref[0] + am_ref[1]) * rm + bm_ref[...]
    rl = 1.0 / (dl_ref[0, :, :1] + dl_ref[1, :, :1] + 1e-16)
    lv = (al_ref[0] + al_ref[1]) * rl + bl_ref[...]
    mu_ref[...] = mu
    lv_ref[...] = lv
    z = eps_ref[...] * jnp.exp(lv) + mu
    h3 = _dot(z, w3_ref[...])
    h3_ref[...] = h3
    a3s_ref[...] = _dot(h3, s3_ref[...])
    a3d_ref[...] = _dot(h3, d3_ref[...])


def _k4_body(acc_ref, den_ref, b_ref, w4_ref, s4_ref, d4_ref,
             h4_ref, a4s_ref, a4d_ref):
    rc = 1.0 / (den_ref[0, :, :1] + den_ref[1, :, :1] + 1e-16)
    dd = jnp.maximum((acc_ref[0] + acc_ref[1]) * rc + b_ref[...], 0.0)
    h4 = _dot(dd, w4_ref[...])
    h4_ref[...] = h4
    a4s_ref[...] = _dot(h4, s4_ref[...])
    a4d_ref[...] = _dot(h4, d4_ref[...])


def _k5_body(acc_ref, den_ref, b_ref, e_ref, out_ref):
    out_ref[...] = _finalize_block(acc_ref, den_ref, b_ref, e_ref, relu=False)


def _row_specs(shapes):
    return [pl.BlockSpec((2, BR) + s[2:], lambda i: (0, i) + (0,) * (len(s) - 2))
            if len(s) == 3 else pl.BlockSpec((BR,) + s[1:], lambda i: (i, 0))
            for s in shapes]


def _full_spec(a):
    nd = a.ndim
    return pl.BlockSpec(a.shape, lambda i, _n=nd: (0,) * _n)


def _att_mats(att_src, att_dst, heads, out_ch):
    f = heads * out_ch
    a_s = att_src[0]
    a_d = att_dst[0]
    fi = jnp.arange(f)[:, None] // out_ch
    hi = jnp.arange(16)[None, :]
    mask = (fi == hi).astype(jnp.float32)
    pad_s = jnp.pad(a_s, ((0, 16 - heads), (0, 0)))
    pad_d = jnp.pad(a_d, ((0, 16 - heads), (0, 0)))
    tile_s = jnp.tile(pad_s.T, (heads, 1))
    tile_d = jnp.tile(pad_d.T, (heads, 1))
    return mask * tile_s, mask * tile_d, mask.T


def kernel(x, edge_index, params, eps):
    f32 = jnp.float32
    xp = jnp.pad(x.astype(f32), ((0, NP - N), (0, 0)))
    epsp = jnp.pad(eps.astype(f32), ((0, NP - N), (0, 0)))
    loop = jnp.arange(N, dtype=jnp.int32)
    padi = jnp.full((ET_PAD - ET,), N, jnp.int32)
    src = jnp.concatenate([edge_index[0].astype(jnp.int32), loop, padi])
    dst = jnp.concatenate([edge_index[1].astype(jnp.int32), loop, padi])
    zf = jnp.zeros((SPT, 128), f32)
    zf32 = jnp.zeros((SPT, 32), f32)
    zh = jnp.zeros((SPT, 16), f32)

    p1 = params["enc1"]
    pm = params["enc2_mu"]
    plv = params["enc2_lv"]
    p3 = params["dec1"]
    p4 = params["dec2"]
    s1, d1, e1 = _att_mats(p1["att_src"], p1["att_dst"], HEADS, HID)
    sm, dm, _ = _att_mats(pm["att_src"], pm["att_dst"], 1, LAT)
    sl, dl, _ = _att_mats(plv["att_src"], plv["att_dst"], 1, LAT)
    s3, d3, _ = _att_mats(p3["att_src"], p3["att_dst"], 1, HID * HEADS)
    s4, d4, e4 = _att_mats(p4["att_src"], p4["att_dst"], HEADS, D_IN // HEADS)

    h1, a1s, a1d = _dense1(xp, p1["W"], s1, d1, 128)
    acc1, den1 = _AGG128(h1, a1s, a1d, src, dst, zf, zh)

    hm, ams, amd, hl, als, ald = pl.pallas_call(
        _k2_body,
        grid=(GRID,),
        in_specs=(
            _row_specs([(2, NP, 128), (2, NP, 16)])
            + [_full_spec(a) for a in
               (p1["bias"][None], e1, pm["W"], sm, dm, plv["W"], sl, dl)]
        ),
        out_specs=[pl.BlockSpec((BR, 32), lambda i: (i, 0)),
                   pl.BlockSpec((BR, 16), lambda i: (i, 0)),
                   pl.BlockSpec((BR, 16), lambda i: (i, 0)),
                   pl.BlockSpec((BR, 32), lambda i: (i, 0)),
                   pl.BlockSpec((BR, 16), lambda i: (i, 0)),
                   pl.BlockSpec((BR, 16), lambda i: (i, 0))],
        out_shape=[jax.ShapeDtypeStruct((NP, 32), f32),
                   jax.ShapeDtypeStruct((NP, 16), f32),
                   jax.ShapeDtypeStruct((NP, 16), f32),
                   jax.ShapeDtypeStruct((NP, 32), f32),
                   jax.ShapeDtypeStruct((NP, 16), f32),
                   jax.ShapeDtypeStruct((NP, 16), f32)],
    )(acc1, den1, p1["bias"][None], e1, pm["W"], sm, dm, plv["W"], sl, dl)

    accm, denm = _AGG32(hm, ams, amd, src, dst, zf32, zh)
    accl, denl = _AGG32(hl, als, ald, src, dst, zf32, zh)

    mu, lv, h3, a3s, a3d = pl.pallas_call(
        _k3_body,
        grid=(GRID,),
        in_specs=(
            _row_specs([(2, NP, 32), (2, NP, 16), (2, NP, 32), (2, NP, 16)])
            + [_full_spec(pm["bias"][None]), _full_spec(plv["bias"][None]),
               pl.BlockSpec((BR, 32), lambda i: (i, 0)),
               _full_spec(p3["W"]), _full_spec(s3), _full_spec(d3)]
        ),
        out_specs=[pl.BlockSpec((BR, 32), lambda i: (i, 0)),
                   pl.BlockSpec((BR, 32), lambda i: (i, 0)),
                   pl.BlockSpec((BR, 128), lambda i: (i, 0)),
                   pl.BlockSpec((BR, 16), lambda i: (i, 0)),
                   pl.BlockSpec((BR, 16), lambda i: (i, 0))],
        out_shape=[jax.ShapeDtypeStruct((NP, 32), f32),
                   jax.ShapeDtypeStruct((NP, 32), f32),
                   jax.ShapeDtypeStruct((NP, 128), f32),
                   jax.ShapeDtypeStruct((NP, 16), f32),
                   jax.ShapeDtypeStruct((NP, 16), f32)],
    )(accm, denm, accl, denl, pm["bias"][None], plv["bias"][None], epsp,
      p3["W"], s3, d3)

    acc3, den3 = _AGG128C(h3, a3s, a3d, src, dst, zf, zh)

    h4, a4s, a4d = pl.pallas_call(
        _k4_body,
        grid=(GRID,),
        in_specs=(
            _row_specs([(2, NP, 128), (2, NP, 16)])
            + [_full_spec(a) for a in (p3["bias"][None], p4["W"], s4, d4)]
        ),
        out_specs=[pl.BlockSpec((BR, 128), lambda i: (i, 0)),
                   pl.BlockSpec((BR, 16), lambda i: (i, 0)),
                   pl.BlockSpec((BR, 16), lambda i: (i, 0))],
        out_shape=[jax.ShapeDtypeStruct((NP, 128), f32),
                   jax.ShapeDtypeStruct((NP, 16), f32),
                   jax.ShapeDtypeStruct((NP, 16), f32)],
    )(acc3, den3, p3["bias"][None], p4["W"], s4, d4)

    acc4, den4 = _AGG128(h4, a4s, a4d, src, dst, zf, zh)

    recon = pl.pallas_call(
        _k5_body,
        grid=(GRID,),
        in_specs=(
            _row_specs([(2, NP, 128), (2, NP, 16)])
            + [_full_spec(p4["bias"][None]), _full_spec(e4)]
        ),
        out_specs=pl.BlockSpec((BR, 128), lambda i: (i, 0)),
        out_shape=jax.ShapeDtypeStruct((NP, 128), f32),
    )(acc4, den4, p4["bias"][None], e4)

    return (recon[:N], mu[:N], lv[:N])
